# Initial kernel scaffold; baseline (speedup 1.0000x reference)
import jax
import jax.numpy as jnp
from jax.experimental import pallas as pl


def kernel(cls_logits, pred_boxes):
    raise NotImplementedError("write your pallas kernel here")


# speedup vs baseline: 1.4581x; 1.4581x over previous
import jax
import jax.numpy as jnp
from jax.experimental import pallas as pl

_NUM_FEATS = 20000
_NUM_CLASSES = 80
_K = 1000
_KP = 1024
_MAX_DETS = 100
_IMG_SIZE = 1024.0
_THR = 0.5


def _score_kernel(logits_ref, boxes_ref, out_ref):
    wd = ((boxes_ref[:, 2:3] > 0.0) & (boxes_ref[:, 3:4] > 0.0)).astype(jnp.float32)
    out_ref[...] = jax.nn.sigmoid(logits_ref[:, :_NUM_CLASSES]) * wd


def _iou_mask_kernel(b_ref, bt_ref, m_ref):
    x1c = b_ref[:, 0:1]
    y1c = b_ref[:, 1:2]
    x2c = b_ref[:, 2:3]
    y2c = b_ref[:, 3:4]
    x1r = bt_ref[0:1, :]
    y1r = bt_ref[1:2, :]
    x2r = bt_ref[2:3, :]
    y2r = bt_ref[3:4, :]
    area_c = (x2c - x1c) * (y2c - y1c)
    area_r = (x2r - x1r) * (y2r - y1r)
    w = jnp.maximum(jnp.minimum(x2c, x2r) - jnp.maximum(x1c, x1r), 0.0)
    h = jnp.maximum(jnp.minimum(y2c, y2r) - jnp.maximum(y1c, y1r), 0.0)
    inter = w * h
    union = area_c + area_r - inter
    iou = inter / jnp.maximum(union, 1e-9)
    m_ref[...] = (iou > _THR).astype(jnp.float32)


def _nms_seq_kernel(m_ref, keep_ref):
    col = jax.lax.broadcasted_iota(jnp.int32, (1, _KP), 1)

    def body(i, keep):
        row = m_ref[pl.ds(i, 1), :]
        keep_i = jnp.sum(jnp.where(col == i, keep, 0.0))
        gt = (col > i).astype(jnp.float32)
        return keep * (1.0 - row * gt * keep_i)

    keep_ref[...] = jax.lax.fori_loop(0, _K, body,
                                      jnp.ones((1, _KP), jnp.float32))


def kernel(cls_logits, pred_boxes):
    scores2d = pl.pallas_call(
        _score_kernel,
        out_shape=jax.ShapeDtypeStruct((_NUM_FEATS, _NUM_CLASSES), jnp.float32),
    )(cls_logits, pred_boxes)
    scores = scores2d.reshape(-1)

    cand_scores, cand_ids = jax.lax.top_k(scores, _K)
    cand_labels = (cand_ids % _NUM_CLASSES).astype(jnp.int32)
    feat_ids = cand_ids // _NUM_CLASSES

    cx = pred_boxes[feat_ids, 0]
    cy = pred_boxes[feat_ids, 1]
    w = pred_boxes[feat_ids, 2]
    h = pred_boxes[feat_ids, 3]
    cand_boxes = jnp.stack(
        [cx - w / 2, cy - h / 2, cx + w / 2, cy + h / 2], axis=1)

    off = cand_labels.astype(jnp.float32)[:, None] * (_IMG_SIZE * 4.0)
    nms_boxes = cand_boxes + off
    pad = jnp.tile(jnp.array([[2e9, 2e9, 2e9 + 1.0, 2e9 + 1.0]], jnp.float32),
                   (_KP - _K, 1))
    nms_p = jnp.concatenate([nms_boxes, pad], axis=0)

    mask = pl.pallas_call(
        _iou_mask_kernel,
        grid=(_KP // 64,),
        in_specs=[
            pl.BlockSpec((64, 4), lambda k: (k, 0)),
            pl.BlockSpec((4, _KP), lambda k: (0, 0)),
        ],
        out_specs=pl.BlockSpec((64, _KP), lambda k: (k, 0)),
        out_shape=jax.ShapeDtypeStruct((_KP, _KP), jnp.float32),
    )(nms_p, nms_p.T)

    keep_f = pl.pallas_call(
        _nms_seq_kernel,
        out_shape=jax.ShapeDtypeStruct((1, _KP), jnp.float32),
    )(mask)
    keep = keep_f[0, :_K] > 0.5

    kept_scores = jnp.where(keep, cand_scores, -1.0)
    _, top_ids = jax.lax.top_k(kept_scores, _MAX_DETS)
    out_labels = cand_labels[top_ids]
    out_boxes = cand_boxes[top_ids]
    out_scores = cand_scores[top_ids]
    batch_ids = jnp.zeros((_MAX_DETS,), dtype=jnp.int32)
    return out_labels, out_boxes, out_scores, batch_ids
